# baseline (speedup 1.0000x reference)
import functools

import jax
import jax.numpy as jnp
from jax import lax
from jax.experimental import pallas as pl
from jax.experimental.pallas import tpu as pltpu
from jax.experimental.pallas import tpu_sc as plsc

D = 32
HIST = 50
BATCH = 16384
NC, NS = 2, 16
NW = NC * NS
BPB = BATCH // NW
BT = BPB // 128
ET = D // 8

_mesh = plsc.VectorSubcoreMesh(core_axis_name="c", subcore_axis_name="s")


@functools.partial(
    pl.kernel,
    mesh=_mesh,
    out_type=jax.ShapeDtypeStruct((HIST, ET, BATCH // 128, 8, 128),
                                  jnp.float32),
    scratch_types=[
        pltpu.VMEM((BPB,), jnp.int32),
        pltpu.VMEM((BPB,), jnp.int32),
        pltpu.VMEM((BPB, D), jnp.float32),
        pltpu.VMEM((BPB, D), jnp.float32),
        pltpu.VMEM((ET, BT, 8, 128), jnp.float32),
        pltpu.VMEM((ET, BT, 8, 128), jnp.float32),
        pltpu.SemaphoreType.DMA,
        pltpu.SemaphoreType.DMA,
        pltpu.SemaphoreType.DMA,
        pltpu.SemaphoreType.DMA,
    ],
    compiler_params=pltpu.CompilerParams(
        use_tc_tiling_on_sc=False, needs_layout_passes=False
    ),
)
def _emb_lookup(xt_hbm, table_hbm, out_hbm, idx0, idx1, rows0, rows1,
                stg0, stg1, g0, g1, o0, o1):
    wid = lax.axis_index("s") * NC + lax.axis_index("c")
    b0 = wid * BPB
    bt0 = wid * BT
    idx = (idx0, idx1)
    rows = (rows0, rows1)
    stg = (stg0, stg1)
    gs = (g0, g1)
    os_ = (o0, o1)

    def load_and_gather(h, b):
        pltpu.sync_copy(xt_hbm.at[h, pl.ds(b0, BPB)], idx[b])
        pltpu.async_copy(table_hbm.at[idx[b]], rows[b], gs[b])

    def wait_gather(b):
        pltpu.make_async_copy(table_hbm.at[idx[b]], rows[b], gs[b]).wait()

    def write_out(h, b):
        pltpu.async_copy(stg[b], out_hbm.at[h, :, pl.ds(bt0, BT)], os_[b])

    def wait_out(h, b):
        pltpu.make_async_copy(stg[b], out_hbm.at[h, :, pl.ds(bt0, BT)],
                              os_[b]).wait()

    def transpose_block(b):
        rows_b = rows[b]
        stg_b = stg[b]
        lane = lax.iota(jnp.int32, 16)

        def tr_body(e, carry):
            col = jnp.full((16,), e, jnp.int32)
            et = e >> 3
            es = e & 7
            for btl in range(BT):
                for g in range(8):
                    row = lane + (btl * 128 + g * 16)
                    v = plsc.load_gather(rows_b, [row, col])
                    stg_b[et, btl, es, pl.ds(g * 16, 16)] = v
            return carry

        lax.fori_loop(0, D, tr_body, 0)

    def step(h, b, first, last):
        wait_gather(b)
        if not first:
            wait_out(h - 2, b)
        transpose_block(b)
        write_out(h, b)
        if not last:
            load_and_gather(h + 2, b)

    load_and_gather(0, 0)
    load_and_gather(1, 1)
    step(0, 0, True, False)
    step(1, 1, True, False)

    def body(t, carry):
        for b in range(2):
            step(2 * t + b, b, False, False)
        return carry

    lax.fori_loop(1, HIST // 2 - 1, body, 0)
    step(HIST - 2, 0, False, True)
    step(HIST - 1, 1, False, True)
    wait_out(HIST - 2, 0)
    wait_out(HIST - 1, 1)


def kernel(x, atom_emb_weight):
    o5 = _emb_lookup(x.T, atom_emb_weight)
    return (
        o5.transpose(0, 1, 3, 2, 4)
        .reshape(HIST, D, BATCH)
        .transpose(2, 0, 1)
    )


# speedup vs baseline: 1.2594x; 1.2594x over previous
import functools

import jax
import jax.numpy as jnp
from jax import lax
from jax.experimental import pallas as pl
from jax.experimental.pallas import tpu as pltpu
from jax.experimental.pallas import tpu_sc as plsc

D = 32
HIST = 50
BATCH = 16384
NC, NS = 2, 16
NW = NC * NS
BPB = BATCH // NW
BT = BPB // 128
ET = D // 8

_mesh = plsc.VectorSubcoreMesh(core_axis_name="c", subcore_axis_name="s")


@functools.partial(
    pl.kernel,
    mesh=_mesh,
    out_type=jax.ShapeDtypeStruct((HIST, ET, BATCH // 128, 8, 128),
                                  jnp.float32),
    scratch_types=[
        pltpu.VMEM((BPB,), jnp.int32),
        pltpu.VMEM((BPB,), jnp.int32),
        pltpu.VMEM((BPB, D), jnp.float32),
        pltpu.VMEM((BPB, D), jnp.float32),
        pltpu.VMEM((ET, BT, 8, 128), jnp.float32),
        pltpu.VMEM((ET, BT, 8, 128), jnp.float32),
        pltpu.SemaphoreType.DMA,
        pltpu.SemaphoreType.DMA,
        pltpu.SemaphoreType.DMA,
        pltpu.SemaphoreType.DMA,
    ],
    compiler_params=pltpu.CompilerParams(
        use_tc_tiling_on_sc=False, needs_layout_passes=False
    ),
)
def _emb_lookup(xt_hbm, table_hbm, out_hbm, idx0, idx1, rows0, rows1,
                stg0, stg1, g0, g1, o0, o1):
    wid = lax.axis_index("s") * NC + lax.axis_index("c")
    b0 = wid * BPB
    bt0 = wid * BT
    idx = (idx0, idx1)
    rows = (rows0, rows1)
    stg = (stg0, stg1)
    gs = (g0, g1)
    os_ = (o0, o1)

    def load_and_gather(h, b):
        pltpu.sync_copy(xt_hbm.at[h, pl.ds(b0, BPB)], idx[b])
        pltpu.async_copy(table_hbm.at[idx[b]], rows[b], gs[b])

    def wait_gather(b):
        pltpu.make_async_copy(table_hbm.at[idx[b]], rows[b], gs[b]).wait()

    def write_out(h, b):
        pltpu.async_copy(stg[b], out_hbm.at[h, :, pl.ds(bt0, BT)], os_[b])

    def wait_out(h, b):
        pltpu.make_async_copy(stg[b], out_hbm.at[h, :, pl.ds(bt0, BT)],
                              os_[b]).wait()

    def transpose_block(b):
        rows_b = rows[b]
        stg_b = stg[b]
        lane = lax.iota(jnp.int32, 16)

        @plsc.parallel_loop(0, D * BT * 8, unroll=8)
        def _(t):
            e = t >> 5
            row = lane + ((t & 31) << 4)
            col = jnp.full((16,), e, jnp.int32)
            v = plsc.load_gather(rows_b, [row, col])
            stg_b[t >> 8, (t >> 3) & 3, e & 7, pl.ds((t & 7) * 16, 16)] = v

    def step(h, b, first, last):
        wait_gather(b)
        if not first:
            wait_out(h - 2, b)
        transpose_block(b)
        write_out(h, b)
        if not last:
            load_and_gather(h + 2, b)

    load_and_gather(0, 0)
    load_and_gather(1, 1)
    step(0, 0, True, False)
    step(1, 1, True, False)

    def body(t, carry):
        for b in range(2):
            step(2 * t + b, b, False, False)
        return carry

    lax.fori_loop(1, HIST // 2 - 1, body, 0)
    step(HIST - 2, 0, False, True)
    step(HIST - 1, 1, False, True)
    wait_out(HIST - 2, 0)
    wait_out(HIST - 1, 1)


def kernel(x, atom_emb_weight):
    o5 = _emb_lookup(x.T, atom_emb_weight)
    return (
        o5.transpose(0, 1, 3, 2, 4)
        .reshape(HIST, D, BATCH)
        .transpose(2, 0, 1)
    )


# speedup vs baseline: 1.2669x; 1.0059x over previous
import functools

import jax
import jax.numpy as jnp
from jax import lax
from jax.experimental import pallas as pl
from jax.experimental.pallas import tpu as pltpu
from jax.experimental.pallas import tpu_sc as plsc

D = 32
HIST = 50
BATCH = 16384
NC, NS = 2, 16
NW = NC * NS
BPB = BATCH // NW
BT = BPB // 128
ET = D // 8

_mesh = plsc.VectorSubcoreMesh(core_axis_name="c", subcore_axis_name="s")


@functools.partial(
    pl.kernel,
    mesh=_mesh,
    out_type=jax.ShapeDtypeStruct((HIST, ET, BATCH // 128, 8, 128),
                                  jnp.float32),
    scratch_types=[
        pltpu.VMEM((BPB,), jnp.int32),
        pltpu.VMEM((BPB,), jnp.int32),
        pltpu.VMEM((BPB, D), jnp.float32),
        pltpu.VMEM((BPB, D), jnp.float32),
        pltpu.VMEM((ET, BT, 8, 128), jnp.float32),
        pltpu.VMEM((ET, BT, 8, 128), jnp.float32),
        pltpu.SemaphoreType.DMA,
        pltpu.SemaphoreType.DMA,
        pltpu.SemaphoreType.DMA,
        pltpu.SemaphoreType.DMA,
    ],
    compiler_params=pltpu.CompilerParams(
        use_tc_tiling_on_sc=False, needs_layout_passes=False
    ),
)
def _emb_lookup(xt_hbm, table_hbm, out_hbm, idx0, idx1, rows0, rows1,
                stg0, stg1, g0, g1, o0, o1):
    wid = lax.axis_index("s") * NC + lax.axis_index("c")
    b0 = wid * BPB
    bt0 = wid * BT
    idx = (idx0, idx1)
    rows = (rows0, rows1)
    stg = (stg0, stg1)
    gs = (g0, g1)
    os_ = (o0, o1)

    def load_and_gather(h, b):
        pltpu.sync_copy(xt_hbm.at[h, pl.ds(b0, BPB)], idx[b])
        pltpu.async_copy(table_hbm.at[idx[b]], rows[b], gs[b])

    def wait_gather(b):
        pltpu.make_async_copy(table_hbm.at[idx[b]], rows[b], gs[b]).wait()

    def write_out(h, b):
        pltpu.async_copy(stg[b], out_hbm.at[h, :, pl.ds(bt0, BT)], os_[b])

    def wait_out(h, b):
        pltpu.make_async_copy(stg[b], out_hbm.at[h, :, pl.ds(bt0, BT)],
                              os_[b]).wait()

    def transpose_block(b):
        rows_b = rows[b]
        stg_b = stg[b]
        lane = lax.iota(jnp.int32, 16)

        @plsc.parallel_loop(0, D * BT * 8, unroll=16)
        def _(t):
            e = t >> 5
            row = lane + ((t & 31) << 4)
            col = jnp.full((16,), e, jnp.int32)
            v = plsc.load_gather(rows_b, [row, col])
            stg_b[t >> 8, (t >> 3) & 3, e & 7, pl.ds((t & 7) * 16, 16)] = v

    def step(h, b, first, last):
        wait_gather(b)
        if not first:
            wait_out(h - 2, b)
        transpose_block(b)
        write_out(h, b)
        if not last:
            load_and_gather(h + 2, b)

    load_and_gather(0, 0)
    load_and_gather(1, 1)
    step(0, 0, True, False)
    step(1, 1, True, False)

    def body(t, carry):
        for b in range(2):
            step(2 * t + b, b, False, False)
        return carry

    lax.fori_loop(1, HIST // 2 - 1, body, 0)
    step(HIST - 2, 0, False, True)
    step(HIST - 1, 1, False, True)
    wait_out(HIST - 2, 0)
    wait_out(HIST - 1, 1)


def kernel(x, atom_emb_weight):
    o5 = _emb_lookup(x.T, atom_emb_weight)
    return (
        o5.transpose(0, 1, 3, 2, 4)
        .reshape(HIST, D, BATCH)
        .transpose(2, 0, 1)
    )


# speedup vs baseline: 4.1533x; 3.2783x over previous
import functools

import jax
import jax.numpy as jnp
from jax import lax
from jax.experimental import pallas as pl
from jax.experimental.pallas import tpu as pltpu
from jax.experimental.pallas import tpu_sc as plsc

D = 32
HIST = 50
BATCH = 16384
NC, NS = 2, 16
NW = NC * NS
BPB = BATCH // NW
BT = BPB // 128
ET = D // 8

_mesh = plsc.VectorSubcoreMesh(core_axis_name="c", subcore_axis_name="s")


@functools.partial(
    pl.kernel,
    mesh=_mesh,
    out_type=jax.ShapeDtypeStruct((HIST, ET, BATCH // 128, 8, 128),
                                  jnp.float32),
    scratch_types=[
        pltpu.VMEM((BPB,), jnp.int32),
        pltpu.VMEM((BPB,), jnp.int32),
        pltpu.VMEM((BPB, D), jnp.float32),
        pltpu.VMEM((BPB, D), jnp.float32),
        pltpu.VMEM((D, BPB + 3), jnp.float32),
        pltpu.VMEM((D, BPB + 3), jnp.float32),
        pltpu.SemaphoreType.DMA,
        pltpu.SemaphoreType.DMA,
        pltpu.SemaphoreType.DMA,
        pltpu.SemaphoreType.DMA,
    ],
    compiler_params=pltpu.CompilerParams(
        use_tc_tiling_on_sc=False, needs_layout_passes=False
    ),
)
def _emb_lookup(xt_hbm, table_hbm, out_hbm, idx0, idx1, rows0, rows1,
                stg0, stg1, g0, g1, o0, o1):
    wid = lax.axis_index("s") * NC + lax.axis_index("c")
    b0 = wid * BPB
    bt0 = wid * BT
    idx = (idx0, idx1)
    rows = (rows0, rows1)
    stg = (stg0, stg1)
    gs = (g0, g1)
    os_ = (o0, o1)

    def load_and_gather(h, b):
        pltpu.sync_copy(xt_hbm.at[h, pl.ds(b0, BPB)], idx[b])
        pltpu.async_copy(table_hbm.at[idx[b]], rows[b], gs[b])

    def wait_gather(b):
        pltpu.make_async_copy(table_hbm.at[idx[b]], rows[b], gs[b]).wait()

    def write_out(h, b):
        for et in range(ET):
            for btl in range(BT):
                pltpu.async_copy(
                    stg[b].at[pl.ds(et * 8, 8), pl.ds(btl * 128, 128)],
                    out_hbm.at[h, et, bt0 + btl],
                    os_[b],
                )

    def wait_out(h, b):
        for et in range(ET):
            for btl in range(BT):
                pltpu.make_async_copy(
                    stg[b].at[pl.ds(et * 8, 8), pl.ds(btl * 128, 128)],
                    out_hbm.at[h, et, bt0 + btl],
                    os_[b],
                ).wait()

    def transpose_block(b):
        rows_b = rows[b]
        stg_b = stg[b]
        lane = lax.iota(jnp.int32, 16)

        @plsc.parallel_loop(0, BPB * 2, unroll=16)
        def _(t):
            bb = t >> 1
            eoff = (t & 1) * 16
            v = rows_b[bb, pl.ds(eoff, 16)]
            plsc.store_scatter(stg_b, [lane + eoff, jnp.full((16,), bb, jnp.int32)], v)

    def step(h, b, first, last):
        wait_gather(b)
        if not first:
            wait_out(h - 2, b)
        transpose_block(b)
        write_out(h, b)
        if not last:
            load_and_gather(h + 2, b)

    load_and_gather(0, 0)
    load_and_gather(1, 1)
    step(0, 0, True, False)
    step(1, 1, True, False)

    def body(t, carry):
        for b in range(2):
            step(2 * t + b, b, False, False)
        return carry

    lax.fori_loop(1, HIST // 2 - 1, body, 0)
    step(HIST - 2, 0, False, True)
    step(HIST - 1, 1, False, True)
    wait_out(HIST - 2, 0)
    wait_out(HIST - 1, 1)


def kernel(x, atom_emb_weight):
    o5 = _emb_lookup(x.T, atom_emb_weight)
    return (
        o5.transpose(0, 1, 3, 2, 4)
        .reshape(HIST, D, BATCH)
        .transpose(2, 0, 1)
    )
